# baseline (speedup 1.0000x reference)
import functools

import jax
import jax.numpy as jnp
from jax import lax
from jax.experimental import pallas as pl
from jax.experimental.pallas import tpu as pltpu
from jax.experimental.pallas import tpu_sc as plsc

N = 10000
E = 320000
H = 128
G = 128

_NC = 2
_NS = 16
_NW = _NC * _NS
_CH = 80
_NCK = 125
_RPS = 624
_RTAIL = N - _NS * _RPS


def _make_seg_sum():
    mesh = plsc.VectorSubcoreMesh(core_axis_name="c", subcore_axis_name="s")

    @functools.partial(
        pl.kernel,
        mesh=mesh,
        out_type=jax.ShapeDtypeStruct((_NC, N, H), jnp.float32),
        scratch_types=[
            pltpu.VMEM((_CH,), jnp.int32),
            pltpu.VMEM((_CH,), jnp.int32),
            pltpu.VMEM((_CH,), jnp.int32),
            pltpu.VMEM((_CH,), jnp.int32),
            pltpu.VMEM((_CH,), jnp.int32),
            pltpu.VMEM((_CH,), jnp.int32),
            pltpu.VMEM((_CH,), jnp.int32),
            pltpu.VMEM((_CH,), jnp.int32),
            pltpu.VMEM((_CH, H), jnp.float32),
            pltpu.VMEM((_CH, H), jnp.float32),
            pltpu.VMEM((_CH, H), jnp.float32),
            pltpu.VMEM((_CH, H), jnp.float32),
            pltpu.VMEM_SHARED((N, H), jnp.float32),
            pltpu.SemaphoreType.DMA,
            pltpu.SemaphoreType.DMA,
            pltpu.SemaphoreType.DMA,
            pltpu.SemaphoreType.DMA,
            pltpu.SemaphoreType.DMA,
            pltpu.SemaphoreType.DMA,
            pltpu.SemaphoreType.DMA,
            pltpu.SemaphoreType.DMA,
        ],
    )
    def seg(h_hbm, src3_hbm, dst3_hbm, out_hbm,
            sidx0, sidx1, sidx2, sidx3, didx0, didx1, didx2, didx3,
            rows0, rows1, rows2, rows3, acc,
            semg0, semg1, semg2, semg3, semd0, semd1, semd2, semd3):
        cid = lax.axis_index("c")
        sid = lax.axis_index("s")
        wid = cid * _NS + sid
        r0 = sid * _RPS
        c3 = pltpu.async_copy(h_hbm.at[pl.ds(r0, _RPS)],
                              acc.at[pl.ds(r0, _RPS)], semg0)

        @pl.when(sid == 0)
        def _():
            pltpu.async_copy(h_hbm.at[pl.ds(_NS * _RPS, _RTAIL)],
                             acc.at[pl.ds(_NS * _RPS, _RTAIL)], semg1).wait()

        c3.wait()
        plsc.subcore_barrier()

        ebase = wid * _NCK * _CH
        bufs = ((sidx0, didx0, rows0, semg0, semd0),
                (sidx1, didx1, rows1, semg1, semd1),
                (sidx2, didx2, rows2, semg2, semd2),
                (sidx3, didx3, rows3, semg3, semd3))

        def issue_idx(c, p):
            si, di, ro, sg, sd = bufs[p]
            pltpu.async_copy(src3_hbm.at[pl.ds(ebase + c * _CH, _CH)], si, sd)
            pltpu.async_copy(dst3_hbm.at[pl.ds(ebase + c * _CH, _CH)], di, sd)

        def issue_gather(c, p):
            si, di, ro, sg, sd = bufs[p]
            pltpu.make_async_copy(src3_hbm.at[pl.ds(ebase + c * _CH, _CH)], si, sd).wait()
            pltpu.make_async_copy(dst3_hbm.at[pl.ds(ebase + c * _CH, _CH)], di, sd).wait()
            pltpu.async_copy(h_hbm.at[si], ro, sg)

        def drain(c, p):
            si, di, ro, sg, sd = bufs[p]
            pltpu.make_async_copy(h_hbm.at[si], ro, sg).wait()
            pltpu.sync_copy(ro, acc.at[di], add=True)

        for p in range(4):
            issue_idx(p, p)
        for p in range(4):
            issue_gather(p, p)

        def body(j, carry):
            c0 = 4 * j
            for p in range(4):
                drain(c0 + p, p)
                issue_idx(c0 + p + 4, p)
                issue_gather(c0 + p + 4, p)
            return carry

        lax.fori_loop(0, (_NCK - 5) // 4, body, 0)

        for c in range(((_NCK - 5) // 4) * 4, _NCK):
            drain(c, c % 4)
            if c + 4 <= _NCK - 1:
                issue_idx(c + 4, (c + 4) % 4)
                issue_gather(c + 4, (c + 4) % 4)

        plsc.subcore_barrier()
        pltpu.sync_copy(acc.at[pl.ds(r0, _RPS)], out_hbm.at[cid, pl.ds(r0, _RPS)])

        @pl.when(sid == 0)
        def _():
            pltpu.sync_copy(acc.at[pl.ds(_NS * _RPS, _RTAIL)],
                            out_hbm.at[cid, pl.ds(_NS * _RPS, _RTAIL)])

    return seg


_seg_sum = _make_seg_sum()


_R = 2000


def _proj_body(x_ref, w_ref, b_ref, o_ref):
    acc = jnp.dot(x_ref[...], w_ref[...], preferred_element_type=jnp.float32)
    o_ref[...] = jnp.maximum(acc + b_ref[...], 0.0)


def _proj(x, w, b):
    return pl.pallas_call(
        _proj_body,
        grid=(N // _R,),
        in_specs=[
            pl.BlockSpec((_R, H), lambda i: (i, 0)),
            pl.BlockSpec((H, H), lambda i: (0, 0)),
            pl.BlockSpec((1, H), lambda i: (0, 0)),
        ],
        out_specs=pl.BlockSpec((_R, H), lambda i: (i, 0)),
        out_shape=jax.ShapeDtypeStruct((N, H), jnp.float32),
    )(x, w, b)


def _mlp_body(eps_ref, h_ref, a_ref, w1_ref, b1_ref, w2_ref, b2_ref,
              w3_ref, b3_ref, o_ref, *, residual):
    h = h_ref[...]
    eps = eps_ref[0]
    m = (eps - 1.0) * h + a_ref[0] + a_ref[1]
    m = jnp.dot(m, w1_ref[...], preferred_element_type=jnp.float32) + b1_ref[...]
    m = jnp.maximum(m, 0.0)
    m = jnp.dot(m, w2_ref[...], preferred_element_type=jnp.float32) + b2_ref[...]
    m = jnp.maximum(m, 0.0)
    m = jnp.dot(m, w3_ref[...], preferred_element_type=jnp.float32) + b3_ref[...]
    hn = jnp.maximum(m, 0.0)
    o_ref[...] = h + hn if residual else hn


def _mlp(h, agg2, eps, w1, b1, w2, b2, w3, b3, residual):
    body = functools.partial(_mlp_body, residual=residual)
    return pl.pallas_call(
        body,
        grid=(N // _R,),
        in_specs=[
            pl.BlockSpec(memory_space=pltpu.SMEM),
            pl.BlockSpec((_R, H), lambda i: (i, 0)),
            pl.BlockSpec((_NC, _R, H), lambda i: (0, i, 0)),
            pl.BlockSpec((H, 3 * H), lambda i: (0, 0)),
            pl.BlockSpec((1, 3 * H), lambda i: (0, 0)),
            pl.BlockSpec((3 * H, 2 * H), lambda i: (0, 0)),
            pl.BlockSpec((1, 2 * H), lambda i: (0, 0)),
            pl.BlockSpec((2 * H, H), lambda i: (0, 0)),
            pl.BlockSpec((1, H), lambda i: (0, 0)),
        ],
        out_specs=pl.BlockSpec((_R, H), lambda i: (i, 0)),
        out_shape=jax.ShapeDtypeStruct((N, H), jnp.float32),
    )(eps, h, agg2, w1, b1, w2, b2, w3, b3)


def _pool_body(h_ref, b_ref, w1_ref, b1_ref, w2_ref, b2_ref, w3_ref, b3_ref,
               o_ref, pmax_ref):
    h = h_ref[...]
    bid = b_ref[...]
    gids = lax.broadcasted_iota(jnp.int32, (N, G), 1)
    oh = (bid == gids).astype(jnp.float32)
    psum = lax.dot_general(oh, h, (((0,), (0,)), ((), ())),
                           preferred_element_type=jnp.float32)
    cnt = jnp.sum(oh, axis=0)[:, None]
    pmean = psum / jnp.maximum(cnt, 1.0)

    def body(g, carry):
        mask = bid == g
        hm = jnp.where(mask, h, -jnp.inf)
        row = jnp.max(hm, axis=0)
        pmax_ref[pl.ds(g, 1), :] = row[None, :]
        return carry

    lax.fori_loop(0, G, body, 0)
    pmax = pmax_ref[...]
    pmax = jnp.where(jnp.isfinite(pmax), pmax, 0.0)
    ge = jnp.concatenate([psum, pmean, pmax], axis=1)
    f = jnp.dot(ge, w1_ref[...], preferred_element_type=jnp.float32) + b1_ref[...]
    f = jnp.maximum(f, 0.0)
    f = jnp.dot(f, w2_ref[...], preferred_element_type=jnp.float32) + b2_ref[...]
    f = jnp.maximum(f, 0.0)
    o_ref[...] = jnp.dot(f, w3_ref[...], preferred_element_type=jnp.float32) + b3_ref[...]


def _pool_cls(h, batch2d, w1, b1, w2, b2, w3, b3):
    return pl.pallas_call(
        _pool_body,
        out_shape=jax.ShapeDtypeStruct((G, 6), jnp.float32),
        scratch_shapes=[pltpu.VMEM((G, H), jnp.float32)],
    )(h, batch2d, w1, b1, w2, b2, w3, b3)


def kernel(x, edge_index, batch, params):
    inv_s = 1.0 / jnp.sqrt(jnp.float32(1.0) + 1e-5)

    def fold(w, b, g, be):
        scale = g * inv_s
        return w * scale[None, :], (b * scale + be)[None, :]

    src3 = edge_index[0]
    dst3 = edge_index[1]

    w_in, b_in = fold(params['in_W'], params['in_b'], params['in_g'], params['in_be'])
    h = _proj(x, w_in, b_in)

    for i in range(5):
        w1, b1 = fold(params['c%d_W1' % i], params['c%d_b1' % i],
                      params['c%d_g1' % i], params['c%d_be1' % i])
        w2, b2 = fold(params['c%d_W2' % i], params['c%d_b2' % i],
                      params['c%d_g2' % i], params['c%d_be2' % i])
        w3, b3 = fold(params['c%d_W3' % i], params['c%d_b3' % i],
                      params['n%d_g' % i], params['n%d_be' % i])
        eps = params['c%d_eps' % i].reshape(1)
        agg2 = _seg_sum(h, src3, dst3)
        h = _mlp(h, agg2, eps, w1, b1, w2, b2, w3, b3, residual=(i > 0))

    wc1, bc1 = fold(params['cl_W1'], params['cl_b1'], params['cl_g1'], params['cl_be1'])
    wc2, bc2 = fold(params['cl_W2'], params['cl_b2'], params['cl_g2'], params['cl_be2'])
    fw = params['f_W']
    fb = params['f_b'][None, :]
    return _pool_cls(h, batch.reshape(N, 1), wc1, bc1, wc2, bc2, fw, fb)


# speedup vs baseline: 1.2647x; 1.2647x over previous
import functools

import jax
import jax.numpy as jnp
from jax import lax
from jax.experimental import pallas as pl
from jax.experimental.pallas import tpu as pltpu
from jax.experimental.pallas import tpu_sc as plsc

N = 10000
E = 320000
H = 128
G = 128

_NC = 2
_NS = 16
_NW = _NC * _NS
_CH = 80
_NCK = 125
_RPS = 624
_RTAIL = N - _NS * _RPS


def _make_seg_sum():
    mesh = plsc.VectorSubcoreMesh(core_axis_name="c", subcore_axis_name="s")

    @functools.partial(
        pl.kernel,
        mesh=mesh,
        out_type=jax.ShapeDtypeStruct((_NC, N, H), jnp.float32),
        scratch_types=[
            pltpu.VMEM((_NCK * _CH,), jnp.int32),
            pltpu.VMEM((_CH,), jnp.int32),
            pltpu.VMEM((_CH,), jnp.int32),
            pltpu.VMEM((_CH,), jnp.int32),
            pltpu.VMEM((_CH, H), jnp.float32),
            pltpu.VMEM((_CH, H), jnp.float32),
            pltpu.VMEM((_CH, H), jnp.float32),
            pltpu.VMEM_SHARED((N, H), jnp.float32),
            pltpu.SemaphoreType.DMA,
            pltpu.SemaphoreType.DMA,
            pltpu.SemaphoreType.DMA,
            pltpu.SemaphoreType.DMA,
            pltpu.SemaphoreType.DMA,
            pltpu.SemaphoreType.DMA,
        ],
    )
    def seg(h_hbm, src3_hbm, dst3_hbm, out_hbm,
            sidx, didx0, didx1, didx2, rows0, rows1, rows2, acc,
            semg0, semg1, semg2, semd0, semd1, semd2):
        cid = lax.axis_index("c")
        sid = lax.axis_index("s")
        wid = cid * _NS + sid
        r0 = sid * _RPS
        c1 = pltpu.async_copy(src3_hbm.at[pl.ds(wid * _NCK * _CH, _NCK * _CH)],
                              sidx, semg0)
        c2 = pltpu.async_copy(dst3_hbm.at[pl.ds(wid * _NCK * _CH, _CH)],
                              didx0, semg1)
        c3 = pltpu.async_copy(h_hbm.at[pl.ds(r0, _RPS)],
                              acc.at[pl.ds(r0, _RPS)], semg0)

        @pl.when(sid == 0)
        def _():
            pltpu.async_copy(h_hbm.at[pl.ds(_NS * _RPS, _RTAIL)],
                             acc.at[pl.ds(_NS * _RPS, _RTAIL)], semg1).wait()

        c1.wait()
        c2.wait()
        c3.wait()
        plsc.subcore_barrier()

        ebase = wid * _NCK * _CH
        bufs = ((didx0, rows0, semg0, semd0),
                (didx1, rows1, semg1, semd1),
                (didx2, rows2, semg2, semd2))

        def issue(c, p):
            di, ro, sg, sd = bufs[p]
            pltpu.async_copy(dst3_hbm.at[pl.ds(ebase + c * _CH, _CH)], di, sd)
            pltpu.async_copy(h_hbm.at[sidx.at[pl.ds(c * _CH, _CH)]], ro, sg)

        def drain(c, p):
            di, ro, sg, sd = bufs[p]
            pltpu.make_async_copy(dst3_hbm.at[pl.ds(ebase + c * _CH, _CH)], di, sd).wait()
            pltpu.make_async_copy(h_hbm.at[sidx.at[pl.ds(c * _CH, _CH)]], ro, sg).wait()
            pltpu.sync_copy(ro, acc.at[di], add=True)

        for p in range(3):
            issue(p, p)

        def body(j, carry):
            c0 = 3 * j
            for p in range(3):
                drain(c0 + p, p)
                issue(c0 + p + 3, p)
            return carry

        lax.fori_loop(0, (_NCK - 5) // 3, body, 0)

        for i in range(5):
            c = _NCK - 5 + i
            drain(c, c % 3)
            if c + 3 <= _NCK - 1:
                issue(c + 3, (c + 3) % 3)

        plsc.subcore_barrier()
        pltpu.sync_copy(acc.at[pl.ds(r0, _RPS)], out_hbm.at[cid, pl.ds(r0, _RPS)])

        @pl.when(sid == 0)
        def _():
            pltpu.sync_copy(acc.at[pl.ds(_NS * _RPS, _RTAIL)],
                            out_hbm.at[cid, pl.ds(_NS * _RPS, _RTAIL)])

    return seg


_seg_sum = _make_seg_sum()


_R = 2000


def _proj_body(x_ref, w_ref, b_ref, o_ref):
    acc = jnp.dot(x_ref[...], w_ref[...], preferred_element_type=jnp.float32)
    o_ref[...] = jnp.maximum(acc + b_ref[...], 0.0)


def _proj(x, w, b):
    return pl.pallas_call(
        _proj_body,
        grid=(N // _R,),
        in_specs=[
            pl.BlockSpec((_R, H), lambda i: (i, 0)),
            pl.BlockSpec((H, H), lambda i: (0, 0)),
            pl.BlockSpec((1, H), lambda i: (0, 0)),
        ],
        out_specs=pl.BlockSpec((_R, H), lambda i: (i, 0)),
        out_shape=jax.ShapeDtypeStruct((N, H), jnp.float32),
    )(x, w, b)


def _mlp_body(eps_ref, h_ref, a_ref, w1_ref, b1_ref, w2_ref, b2_ref,
              w3_ref, b3_ref, o_ref, *, residual):
    h = h_ref[...]
    eps = eps_ref[0]
    m = (eps - 1.0) * h + a_ref[0] + a_ref[1]
    m = jnp.dot(m, w1_ref[...], preferred_element_type=jnp.float32) + b1_ref[...]
    m = jnp.maximum(m, 0.0)
    m = jnp.dot(m, w2_ref[...], preferred_element_type=jnp.float32) + b2_ref[...]
    m = jnp.maximum(m, 0.0)
    m = jnp.dot(m, w3_ref[...], preferred_element_type=jnp.float32) + b3_ref[...]
    hn = jnp.maximum(m, 0.0)
    o_ref[...] = h + hn if residual else hn


def _mlp(h, agg2, eps, w1, b1, w2, b2, w3, b3, residual):
    body = functools.partial(_mlp_body, residual=residual)
    return pl.pallas_call(
        body,
        grid=(N // _R,),
        in_specs=[
            pl.BlockSpec(memory_space=pltpu.SMEM),
            pl.BlockSpec((_R, H), lambda i: (i, 0)),
            pl.BlockSpec((_NC, _R, H), lambda i: (0, i, 0)),
            pl.BlockSpec((H, 3 * H), lambda i: (0, 0)),
            pl.BlockSpec((1, 3 * H), lambda i: (0, 0)),
            pl.BlockSpec((3 * H, 2 * H), lambda i: (0, 0)),
            pl.BlockSpec((1, 2 * H), lambda i: (0, 0)),
            pl.BlockSpec((2 * H, H), lambda i: (0, 0)),
            pl.BlockSpec((1, H), lambda i: (0, 0)),
        ],
        out_specs=pl.BlockSpec((_R, H), lambda i: (i, 0)),
        out_shape=jax.ShapeDtypeStruct((N, H), jnp.float32),
    )(eps, h, agg2, w1, b1, w2, b2, w3, b3)


def _pool_body(h_ref, b_ref, w1_ref, b1_ref, w2_ref, b2_ref, w3_ref, b3_ref,
               o_ref, pmax_ref):
    h = h_ref[...]
    bid = b_ref[...]
    gids = lax.broadcasted_iota(jnp.int32, (N, G), 1)
    oh = (bid == gids).astype(jnp.float32)
    psum = lax.dot_general(oh, h, (((0,), (0,)), ((), ())),
                           preferred_element_type=jnp.float32)
    cnt = jnp.sum(oh, axis=0)[:, None]
    pmean = psum / jnp.maximum(cnt, 1.0)

    def body(g, carry):
        mask = bid == g
        hm = jnp.where(mask, h, -jnp.inf)
        row = jnp.max(hm, axis=0)
        pmax_ref[pl.ds(g, 1), :] = row[None, :]
        return carry

    lax.fori_loop(0, G, body, 0)
    pmax = pmax_ref[...]
    pmax = jnp.where(jnp.isfinite(pmax), pmax, 0.0)
    ge = jnp.concatenate([psum, pmean, pmax], axis=1)
    f = jnp.dot(ge, w1_ref[...], preferred_element_type=jnp.float32) + b1_ref[...]
    f = jnp.maximum(f, 0.0)
    f = jnp.dot(f, w2_ref[...], preferred_element_type=jnp.float32) + b2_ref[...]
    f = jnp.maximum(f, 0.0)
    o_ref[...] = jnp.dot(f, w3_ref[...], preferred_element_type=jnp.float32) + b3_ref[...]


def _pool_cls(h, batch2d, w1, b1, w2, b2, w3, b3):
    return pl.pallas_call(
        _pool_body,
        out_shape=jax.ShapeDtypeStruct((G, 6), jnp.float32),
        scratch_shapes=[pltpu.VMEM((G, H), jnp.float32)],
    )(h, batch2d, w1, b1, w2, b2, w3, b3)


def kernel(x, edge_index, batch, params):
    inv_s = 1.0 / jnp.sqrt(jnp.float32(1.0) + 1e-5)

    def fold(w, b, g, be):
        scale = g * inv_s
        return w * scale[None, :], (b * scale + be)[None, :]

    src3 = edge_index[0]
    dst3 = edge_index[1]

    w_in, b_in = fold(params['in_W'], params['in_b'], params['in_g'], params['in_be'])
    h = _proj(x, w_in, b_in)

    for i in range(5):
        w1, b1 = fold(params['c%d_W1' % i], params['c%d_b1' % i],
                      params['c%d_g1' % i], params['c%d_be1' % i])
        w2, b2 = fold(params['c%d_W2' % i], params['c%d_b2' % i],
                      params['c%d_g2' % i], params['c%d_be2' % i])
        w3, b3 = fold(params['c%d_W3' % i], params['c%d_b3' % i],
                      params['n%d_g' % i], params['n%d_be' % i])
        eps = params['c%d_eps' % i].reshape(1)
        agg2 = _seg_sum(h, src3, dst3)
        h = _mlp(h, agg2, eps, w1, b1, w2, b2, w3, b3, residual=(i > 0))

    wc1, bc1 = fold(params['cl_W1'], params['cl_b1'], params['cl_g1'], params['cl_be1'])
    wc2, bc2 = fold(params['cl_W2'], params['cl_b2'], params['cl_g2'], params['cl_be2'])
    fw = params['f_W']
    fb = params['f_b'][None, :]
    return _pool_cls(h, batch.reshape(N, 1), wc1, bc1, wc2, bc2, fw, fb)
